# Initial kernel scaffold; baseline (speedup 1.0000x reference)
import jax
import jax.numpy as jnp
from jax.experimental import pallas as pl


def kernel(hidden_states, weight, bias):
    raise NotImplementedError("write your pallas kernel here")


# speedup vs baseline: 1.0712x; 1.0712x over previous
import functools

import jax
import jax.numpy as jnp
from jax.experimental import pallas as pl

TOP_K = 8
NUM_EXPERTS = 64
D_MODEL = 768
BLOCK_M = 1024


def _gate_kernel(h_ref, w_ref, b_ref, out_w_ref, out_i_ref):
    h = h_ref[...]
    w = w_ref[...]
    logits = jnp.dot(h, w, preferred_element_type=jnp.float32) + b_ref[...]

    m = h.shape[0]
    col = jax.lax.broadcasted_iota(jnp.int32, (m, NUM_EXPERTS), 1)
    kcol = jax.lax.broadcasted_iota(jnp.int32, (m, TOP_K), 1)

    vals = jnp.zeros((m, TOP_K), dtype=jnp.float32)
    idxs = jnp.zeros((m, TOP_K), dtype=jnp.int32)
    neg_inf = jnp.float32(-jnp.inf)
    for k in range(TOP_K):
        vmax = jnp.max(logits, axis=-1, keepdims=True)
        imax = jnp.min(
            jnp.where(logits == vmax, col, NUM_EXPERTS), axis=-1, keepdims=True
        )
        vals = jnp.where(kcol == k, vmax, vals)
        idxs = jnp.where(kcol == k, imax, idxs)
        logits = jnp.where(col == imax, neg_inf, logits)

    e = jnp.exp(vals - vals[:, 0:1])
    out_w_ref[...] = e / jnp.sum(e, axis=-1, keepdims=True)
    out_i_ref[...] = idxs


@functools.partial(jax.jit, static_argnames=())
def kernel(hidden_states, weight, bias):
    n_tokens = hidden_states.shape[0]
    w_t = weight.T
    b = bias.reshape(1, NUM_EXPERTS)
    grid = (n_tokens // BLOCK_M,)
    out_w, out_i = pl.pallas_call(
        _gate_kernel,
        grid=grid,
        in_specs=[
            pl.BlockSpec((BLOCK_M, D_MODEL), lambda i: (i, 0)),
            pl.BlockSpec((D_MODEL, NUM_EXPERTS), lambda i: (0, 0)),
            pl.BlockSpec((1, NUM_EXPERTS), lambda i: (0, 0)),
        ],
        out_specs=[
            pl.BlockSpec((BLOCK_M, TOP_K), lambda i: (i, 0)),
            pl.BlockSpec((BLOCK_M, TOP_K), lambda i: (i, 0)),
        ],
        out_shape=[
            jax.ShapeDtypeStruct((n_tokens, TOP_K), jnp.float32),
            jax.ShapeDtypeStruct((n_tokens, TOP_K), jnp.int32),
        ],
    )(hidden_states, w_t, b)
    return out_w, out_i


# speedup vs baseline: 1.4549x; 1.3583x over previous
import functools

import jax
import jax.numpy as jnp
from jax.experimental import pallas as pl

TOP_K = 8
NUM_EXPERTS = 64
D_MODEL = 768
BLOCK_M = 1024


def _gate_kernel(h_ref, w_ref, b_ref, out_w_ref, out_i_ref):
    h = h_ref[...]
    w = w_ref[...]
    logits = jnp.dot(h, w, preferred_element_type=jnp.float32) + b_ref[...]

    m = h.shape[0]
    col = jax.lax.broadcasted_iota(jnp.int32, (m, NUM_EXPERTS), 1).astype(
        jnp.float32
    )
    kcol = jax.lax.broadcasted_iota(jnp.int32, (m, TOP_K), 1)

    vals = jnp.zeros((m, TOP_K), dtype=jnp.float32)
    idxs = jnp.zeros((m, TOP_K), dtype=jnp.float32)
    neg_inf = jnp.float32(-jnp.inf)
    big = jnp.float32(NUM_EXPERTS)
    for k in range(TOP_K):
        vmax = jnp.max(logits, axis=-1, keepdims=True)
        imax = jnp.min(
            jnp.where(logits == vmax, col, big), axis=-1, keepdims=True
        )
        vals = jnp.where(kcol == k, vmax, vals)
        idxs = jnp.where(kcol == k, imax, idxs)
        logits = jnp.where(col == imax, neg_inf, logits)

    e = jnp.exp(vals - vals[:, 0:1])
    out_w_ref[...] = e / jnp.sum(e, axis=-1, keepdims=True)
    out_i_ref[...] = idxs.astype(jnp.int32)


@functools.partial(jax.jit, static_argnames=())
def kernel(hidden_states, weight, bias):
    n_tokens = hidden_states.shape[0]
    w_t = weight.T
    b = bias.reshape(1, NUM_EXPERTS)
    grid = (n_tokens // BLOCK_M,)
    out_w, out_i = pl.pallas_call(
        _gate_kernel,
        grid=grid,
        in_specs=[
            pl.BlockSpec((BLOCK_M, D_MODEL), lambda i: (i, 0)),
            pl.BlockSpec((D_MODEL, NUM_EXPERTS), lambda i: (0, 0)),
            pl.BlockSpec((1, NUM_EXPERTS), lambda i: (0, 0)),
        ],
        out_specs=[
            pl.BlockSpec((BLOCK_M, TOP_K), lambda i: (i, 0)),
            pl.BlockSpec((BLOCK_M, TOP_K), lambda i: (i, 0)),
        ],
        out_shape=[
            jax.ShapeDtypeStruct((n_tokens, TOP_K), jnp.float32),
            jax.ShapeDtypeStruct((n_tokens, TOP_K), jnp.int32),
        ],
    )(hidden_states, w_t, b)
    return out_w, out_i


# speedup vs baseline: 1.5151x; 1.0413x over previous
import functools

import jax
import jax.numpy as jnp
from jax.experimental import pallas as pl

TOP_K = 8
NUM_EXPERTS = 64
D_MODEL = 768
BLOCK_M = 2048


def _gate_kernel(h_ref, w_ref, b_ref, out_w_ref, out_i_ref):
    h = h_ref[...]
    w = w_ref[...]
    logits = jnp.dot(h, w, preferred_element_type=jnp.float32) + b_ref[...]

    m = h.shape[0]
    col = jax.lax.broadcasted_iota(jnp.int32, (m, NUM_EXPERTS), 1).astype(
        jnp.float32
    )
    kcol = jax.lax.broadcasted_iota(jnp.int32, (m, TOP_K), 1)

    vals = jnp.zeros((m, TOP_K), dtype=jnp.float32)
    idxs = jnp.zeros((m, TOP_K), dtype=jnp.float32)
    neg_inf = jnp.float32(-jnp.inf)
    big = jnp.float32(NUM_EXPERTS)
    for k in range(TOP_K):
        vmax = jnp.max(logits, axis=-1, keepdims=True)
        imax = jnp.min(
            jnp.where(logits == vmax, col, big), axis=-1, keepdims=True
        )
        vals = jnp.where(kcol == k, vmax, vals)
        idxs = jnp.where(kcol == k, imax, idxs)
        logits = jnp.where(col == imax, neg_inf, logits)

    e = jnp.exp(vals - vals[:, 0:1])
    out_w_ref[...] = e / jnp.sum(e, axis=-1, keepdims=True)
    out_i_ref[...] = idxs.astype(jnp.int32)


@functools.partial(jax.jit, static_argnames=())
def kernel(hidden_states, weight, bias):
    n_tokens = hidden_states.shape[0]
    w_t = weight.T
    b = bias.reshape(1, NUM_EXPERTS)
    grid = (n_tokens // BLOCK_M,)
    out_w, out_i = pl.pallas_call(
        _gate_kernel,
        grid=grid,
        in_specs=[
            pl.BlockSpec((BLOCK_M, D_MODEL), lambda i: (i, 0)),
            pl.BlockSpec((D_MODEL, NUM_EXPERTS), lambda i: (0, 0)),
            pl.BlockSpec((1, NUM_EXPERTS), lambda i: (0, 0)),
        ],
        out_specs=[
            pl.BlockSpec((BLOCK_M, TOP_K), lambda i: (i, 0)),
            pl.BlockSpec((BLOCK_M, TOP_K), lambda i: (i, 0)),
        ],
        out_shape=[
            jax.ShapeDtypeStruct((n_tokens, TOP_K), jnp.float32),
            jax.ShapeDtypeStruct((n_tokens, TOP_K), jnp.int32),
        ],
    )(hidden_states, w_t, b)
    return out_w, out_i


# speedup vs baseline: 1.5280x; 1.0085x over previous
import functools

import jax
import jax.numpy as jnp
from jax.experimental import pallas as pl

TOP_K = 8
NUM_EXPERTS = 64
D_MODEL = 768
BLOCK_M = 4096


def _gate_kernel(h_ref, w_ref, b_ref, out_w_ref, out_i_ref):
    h = h_ref[...]
    w = w_ref[...]
    logits = jnp.dot(h, w, preferred_element_type=jnp.float32) + b_ref[...]

    m = h.shape[0]
    col = jax.lax.broadcasted_iota(jnp.int32, (m, NUM_EXPERTS), 1).astype(
        jnp.float32
    )
    kcol = jax.lax.broadcasted_iota(jnp.int32, (m, TOP_K), 1)

    vals = jnp.zeros((m, TOP_K), dtype=jnp.float32)
    idxs = jnp.zeros((m, TOP_K), dtype=jnp.float32)
    neg_inf = jnp.float32(-jnp.inf)
    big = jnp.float32(NUM_EXPERTS)
    for k in range(TOP_K):
        vmax = jnp.max(logits, axis=-1, keepdims=True)
        imax = jnp.min(
            jnp.where(logits == vmax, col, big), axis=-1, keepdims=True
        )
        vals = jnp.where(kcol == k, vmax, vals)
        idxs = jnp.where(kcol == k, imax, idxs)
        logits = jnp.where(col == imax, neg_inf, logits)

    e = jnp.exp(vals - vals[:, 0:1])
    out_w_ref[...] = e / jnp.sum(e, axis=-1, keepdims=True)
    out_i_ref[...] = idxs.astype(jnp.int32)


@functools.partial(jax.jit, static_argnames=())
def kernel(hidden_states, weight, bias):
    n_tokens = hidden_states.shape[0]
    w_t = weight.T
    b = bias.reshape(1, NUM_EXPERTS)
    grid = (n_tokens // BLOCK_M,)
    out_w, out_i = pl.pallas_call(
        _gate_kernel,
        grid=grid,
        in_specs=[
            pl.BlockSpec((BLOCK_M, D_MODEL), lambda i: (i, 0)),
            pl.BlockSpec((D_MODEL, NUM_EXPERTS), lambda i: (0, 0)),
            pl.BlockSpec((1, NUM_EXPERTS), lambda i: (0, 0)),
        ],
        out_specs=[
            pl.BlockSpec((BLOCK_M, TOP_K), lambda i: (i, 0)),
            pl.BlockSpec((BLOCK_M, TOP_K), lambda i: (i, 0)),
        ],
        out_shape=[
            jax.ShapeDtypeStruct((n_tokens, TOP_K), jnp.float32),
            jax.ShapeDtypeStruct((n_tokens, TOP_K), jnp.int32),
        ],
    )(hidden_states, w_t, b)
    return out_w, out_i


# speedup vs baseline: 4.3646x; 2.8565x over previous
import jax
import jax.numpy as jnp
from jax.experimental import pallas as pl

TOP_K = 8
NUM_EXPERTS = 64
D_MODEL = 768
BLOCK_M = 4096


def _gate_kernel(h_ref, w_ref, b_ref, out_w_ref, out_i_ref):
    h = h_ref[...]
    w = w_ref[...]
    logits = jax.lax.dot_general(
        w, h, (((1,), (1,)), ((), ())), preferred_element_type=jnp.float32
    ) + b_ref[...]

    m = h.shape[0]
    row = jax.lax.broadcasted_iota(jnp.int32, (NUM_EXPERTS, m), 0).astype(
        jnp.float32
    )
    krow = jax.lax.broadcasted_iota(jnp.int32, (TOP_K, m), 0)

    vals = jnp.zeros((TOP_K, m), dtype=jnp.float32)
    idxs = jnp.zeros((TOP_K, m), dtype=jnp.float32)
    neg_inf = jnp.float32(-jnp.inf)
    big = jnp.float32(NUM_EXPERTS)
    for k in range(TOP_K):
        vmax = jnp.max(logits, axis=0, keepdims=True)
        imax = jnp.min(
            jnp.where(logits == vmax, row, big), axis=0, keepdims=True
        )
        vals = jnp.where(krow == k, vmax, vals)
        idxs = jnp.where(krow == k, imax, idxs)
        logits = jnp.where(row == imax, neg_inf, logits)

    e = jnp.exp(vals - vals[0:1, :])
    out_w_ref[...] = e / jnp.sum(e, axis=0, keepdims=True)
    out_i_ref[...] = idxs.astype(jnp.int32)


def kernel(hidden_states, weight, bias):
    n_tokens = hidden_states.shape[0]
    b = bias.reshape(NUM_EXPERTS, 1)
    grid = (n_tokens // BLOCK_M,)
    out_w, out_i = pl.pallas_call(
        _gate_kernel,
        grid=grid,
        in_specs=[
            pl.BlockSpec((BLOCK_M, D_MODEL), lambda i: (i, 0)),
            pl.BlockSpec((NUM_EXPERTS, D_MODEL), lambda i: (0, 0)),
            pl.BlockSpec((NUM_EXPERTS, 1), lambda i: (0, 0)),
        ],
        out_specs=[
            pl.BlockSpec((TOP_K, BLOCK_M), lambda i: (0, i)),
            pl.BlockSpec((TOP_K, BLOCK_M), lambda i: (0, i)),
        ],
        out_shape=[
            jax.ShapeDtypeStruct((TOP_K, n_tokens), jnp.float32),
            jax.ShapeDtypeStruct((TOP_K, n_tokens), jnp.int32),
        ],
    )(hidden_states, weight, b)
    return out_w.T, out_i.T
